# baseline (speedup 1.0000x reference)
import functools
import math

import jax
import jax.numpy as jnp
import numpy as np
from jax import lax
from jax.experimental import pallas as pl
from jax.experimental.pallas import tpu as pltpu
from jax.experimental.pallas import tpu_sc as plsc

_SIZE = 32000
_PAD = 0
_SMOOTHING = 0.1
_CONF = 1.0 - _SMOOTHING
_N = 4096

_EPS = float(np.float32(_SMOOTHING / (_SIZE - 2)))
_K0 = (_SIZE - 2) * _EPS * math.log(_EPS) + _CONF * math.log(_CONF)

_NC, _NS, _L = 2, 16, 16
_NW = _NC * _NS
_BPW = _N // _NW


@functools.lru_cache(maxsize=None)
def _make_sc_gather():
    mesh = plsc.VectorSubcoreMesh(
        core_axis_name="c", subcore_axis_name="s", num_cores=_NC, num_subcores=_NS
    )

    @functools.partial(
        pl.kernel,
        out_type=jax.ShapeDtypeStruct((_N,), jnp.float32),
        mesh=mesh,
        compiler_params=pltpu.CompilerParams(
            use_tc_tiling_on_sc=True, needs_layout_passes=False
        ),
        scratch_types=[
            pltpu.VMEM((_BPW,), jnp.int32),
            pltpu.VMEM((_BPW // 2, 8, 128), jnp.float32),
            pltpu.VMEM((_BPW,), jnp.float32),
            pltpu.SemaphoreType.DMA,
        ],
    )
    def _sc_gather(x_hbm, tgt_hbm, g_hbm, tgt_v, win_v, g_v, sem):
        wid = lax.axis_index("s") * _NC + lax.axis_index("c")
        row0 = wid * _BPW
        pltpu.sync_copy(tgt_hbm.at[pl.ds(row0, _BPW)], tgt_v)

        half = _BPW // 2
        for p in range(2):
            for k in range(half // _L):
                tv = tgt_v[pl.ds(p * half + k * _L, _L)]
                c0v = (tv >> 7) << 7
                for j in range(_L):
                    r = p * half + k * _L + j
                    pltpu.async_copy(
                        x_hbm.at[pl.ds(row0 + (r // 8) * 8, 8),
                                 pl.ds(pl.multiple_of(c0v[j], 128), 128)],
                        win_v.at[k * _L + j],
                        sem,
                    )

            def _drain(j, _):
                pltpu.make_async_copy(
                    x_hbm.at[pl.ds(0, 8), pl.ds(0, 128)], win_v.at[0], sem
                ).wait()
                return 0

            lax.fori_loop(0, half, _drain, 0)

            for k in range(half // _L):
                rows = k * _L + lax.iota(jnp.int32, _L)
                subs = lax.iota(jnp.int32, _L) & 7
                tv = tgt_v[pl.ds(p * half + k * _L, _L)]
                lanes = tv & 127
                g_v[pl.ds(p * half + k * _L, _L)] = plsc.load_gather(
                    win_v, [rows, subs, lanes]
                )
        pltpu.sync_copy(g_v, g_hbm.at[pl.ds(row0, _BPW)])

    return _sc_gather


_BC = 1280
_KC = _BC // 128
_NBLK = _SIZE // _BC


def _tc_body(x_ref, t_ref, g_ref, out_ref, acc_ref):
    j = pl.program_id(0)

    @pl.when(j == 0)
    def _init():
        acc_ref[...] = jnp.zeros_like(acc_ref)

    acc = acc_ref[...]
    for k in range(_KC):
        chunk = x_ref[:, k * 128:(k + 1) * 128]
        if k == 0:
            lane = lax.broadcasted_iota(jnp.int32, (_N, 128), 1)
            chunk = jnp.where((j == 0) & (lane == 0), 0.0, chunk)
        acc = acc + chunk
    acc_ref[...] = acc

    @pl.when(j == _NBLK - 1)
    def _finish():
        rowsum = jnp.sum(acc_ref[...], axis=1, keepdims=True)
        g = g_ref[...]
        valid = t_ref[...] != _PAD
        li = _K0 - _EPS * (rowsum - g) - _CONF * g
        out_ref[0, 0] = jnp.sum(jnp.where(valid, li, 0.0))


_tc_reduce = pl.pallas_call(
    _tc_body,
    grid=(_NBLK,),
    in_specs=[
        pl.BlockSpec((_N, _BC), lambda j: (0, j)),
        pl.BlockSpec((_N, 1), lambda j: (0, 0)),
        pl.BlockSpec((_N, 1), lambda j: (0, 0)),
    ],
    out_specs=pl.BlockSpec((1, 1), lambda j: (0, 0), memory_space=pltpu.SMEM),
    out_shape=jax.ShapeDtypeStruct((1, 1), jnp.float32),
    scratch_shapes=[pltpu.VMEM((_N, 128), jnp.float32)],
)


def kernel(x, target):
    g = _make_sc_gather()(x, target)
    loss = _tc_reduce(x, target.reshape(_N, 1), g.reshape(_N, 1))
    return loss.reshape(())


# speedup vs baseline: 1.0111x; 1.0111x over previous
import functools
import math

import jax
import jax.numpy as jnp
import numpy as np
from jax import lax
from jax.experimental import pallas as pl
from jax.experimental.pallas import tpu as pltpu
from jax.experimental.pallas import tpu_sc as plsc

_SIZE = 32000
_PAD = 0
_SMOOTHING = 0.1
_CONF = 1.0 - _SMOOTHING
_N = 4096

_EPS = float(np.float32(_SMOOTHING / (_SIZE - 2)))
_K0 = (_SIZE - 2) * _EPS * math.log(_EPS) + _CONF * math.log(_CONF)

_NC, _NS, _L = 2, 16, 16
_NW = _NC * _NS
_BPW = _N // _NW


@functools.lru_cache(maxsize=None)
def _make_sc_gather():
    mesh = plsc.VectorSubcoreMesh(
        core_axis_name="c", subcore_axis_name="s", num_cores=_NC, num_subcores=_NS
    )

    @functools.partial(
        pl.kernel,
        out_type=jax.ShapeDtypeStruct((_N,), jnp.float32),
        mesh=mesh,
        compiler_params=pltpu.CompilerParams(
            use_tc_tiling_on_sc=True, needs_layout_passes=False
        ),
        scratch_types=[
            pltpu.VMEM((_BPW,), jnp.int32),
            pltpu.VMEM((_BPW // 2, 8, 128), jnp.float32),
            pltpu.VMEM((_BPW,), jnp.float32),
            pltpu.SemaphoreType.DMA,
        ],
    )
    def _sc_gather(x_hbm, tgt_hbm, g_hbm, tgt_v, win_v, g_v, sem):
        wid = lax.axis_index("s") * _NC + lax.axis_index("c")
        row0 = wid * _BPW
        pltpu.sync_copy(tgt_hbm.at[pl.ds(row0, _BPW)], tgt_v)

        half = _BPW // 2
        for p in range(2):
            for k in range(half // _L):
                tv = tgt_v[pl.ds(p * half + k * _L, _L)]
                c0v = (tv >> 7) << 7
                for j in range(_L):
                    r = p * half + k * _L + j
                    pltpu.async_copy(
                        x_hbm.at[pl.ds(row0 + (r // 8) * 8, 8),
                                 pl.ds(pl.multiple_of(c0v[j], 128), 128)],
                        win_v.at[k * _L + j],
                        sem,
                    )

            def _drain(j, _):
                pltpu.make_async_copy(
                    x_hbm.at[pl.ds(0, 8), pl.ds(0, 128)], win_v.at[0], sem
                ).wait()
                return 0

            lax.fori_loop(0, half, _drain, 0)

            for k in range(half // _L):
                rows = k * _L + lax.iota(jnp.int32, _L)
                subs = lax.iota(jnp.int32, _L) & 7
                tv = tgt_v[pl.ds(p * half + k * _L, _L)]
                lanes = tv & 127
                g_v[pl.ds(p * half + k * _L, _L)] = plsc.load_gather(
                    win_v, [rows, subs, lanes]
                )
        pltpu.sync_copy(g_v, g_hbm.at[pl.ds(row0, _BPW)])

    return _sc_gather


_BC = 1280
_KC = _BC // 128
_NBLK = _SIZE // _BC


def _tc_body(x_ref, out_ref, acc_ref):
    j = pl.program_id(0)

    @pl.when(j == 0)
    def _init():
        acc_ref[...] = jnp.zeros_like(acc_ref)

    acc = acc_ref[...]
    for k in range(_KC):
        chunk = x_ref[:, k * 128:(k + 1) * 128]
        if k == 0:
            lane = lax.broadcasted_iota(jnp.int32, (_N, 128), 1)
            chunk = jnp.where((j == 0) & (lane == 0), 0.0, chunk)
        acc = acc + chunk
    acc_ref[...] = acc

    @pl.when(j == _NBLK - 1)
    def _finish():
        out_ref[...] = jnp.sum(acc_ref[...], axis=1, keepdims=True)


_tc_reduce = pl.pallas_call(
    _tc_body,
    grid=(_NBLK,),
    in_specs=[pl.BlockSpec((_N, _BC), lambda j: (0, j))],
    out_specs=pl.BlockSpec((_N, 1), lambda j: (0, 0)),
    out_shape=jax.ShapeDtypeStruct((_N, 1), jnp.float32),
    scratch_shapes=[pltpu.VMEM((_N, 128), jnp.float32)],
)


def _combine_body(tcs_ref, g_ref, t_ref, out_ref):
    rowsum = tcs_ref[...]
    g = g_ref[...]
    valid = t_ref[...] != _PAD
    li = _K0 - _EPS * (rowsum - g) - _CONF * g
    out_ref[0, 0] = jnp.sum(jnp.where(valid, li, 0.0))


_combine = pl.pallas_call(
    _combine_body,
    in_specs=[
        pl.BlockSpec((_N, 1), lambda: (0, 0)),
        pl.BlockSpec((_N, 1), lambda: (0, 0)),
        pl.BlockSpec((_N, 1), lambda: (0, 0)),
    ],
    out_specs=pl.BlockSpec((1, 1), lambda: (0, 0), memory_space=pltpu.SMEM),
    out_shape=jax.ShapeDtypeStruct((1, 1), jnp.float32),
)


def kernel(x, target):
    g = _make_sc_gather()(x, target)
    tcs = _tc_reduce(x)
    loss = _combine(tcs, g.reshape(_N, 1), target.reshape(_N, 1))
    return loss.reshape(())


# speedup vs baseline: 1.0126x; 1.0015x over previous
import functools
import math

import jax
import jax.numpy as jnp
import numpy as np
from jax import lax
from jax.experimental import pallas as pl
from jax.experimental.pallas import tpu as pltpu
from jax.experimental.pallas import tpu_sc as plsc

_SIZE = 32000
_PAD = 0
_SMOOTHING = 0.1
_CONF = 1.0 - _SMOOTHING
_N = 4096

_EPS = float(np.float32(_SMOOTHING / (_SIZE - 2)))
_K0 = (_SIZE - 2) * _EPS * math.log(_EPS) + _CONF * math.log(_CONF)

_NC, _NS, _L = 2, 16, 16
_NW = _NC * _NS
_BPW = _N // _NW


@functools.lru_cache(maxsize=None)
def _make_sc_gather():
    mesh = plsc.VectorSubcoreMesh(
        core_axis_name="c", subcore_axis_name="s", num_cores=_NC, num_subcores=_NS
    )

    @functools.partial(
        pl.kernel,
        out_type=jax.ShapeDtypeStruct((_N,), jnp.float32),
        mesh=mesh,
        compiler_params=pltpu.CompilerParams(
            use_tc_tiling_on_sc=True, needs_layout_passes=False
        ),
        scratch_types=[
            pltpu.VMEM((_BPW,), jnp.int32),
            pltpu.VMEM((_BPW // 2, 8, 128), jnp.float32),
            pltpu.VMEM((_BPW,), jnp.float32),
            pltpu.SemaphoreType.DMA,
        ],
    )
    def _sc_gather(x_hbm, tgt_hbm, g_hbm, tgt_v, win_v, g_v, sem):
        wid = lax.axis_index("s") * _NC + lax.axis_index("c")
        row0 = wid * _BPW
        pltpu.sync_copy(tgt_hbm.at[pl.ds(row0, _BPW)], tgt_v)

        half = _BPW // 2
        for p in range(2):
            for k in range(half // _L):
                tv = tgt_v[pl.ds(p * half + k * _L, _L)]
                c0v = (tv >> 7) << 7
                for j in range(_L):
                    r = p * half + k * _L + j
                    pltpu.async_copy(
                        x_hbm.at[pl.ds(row0 + (r // 8) * 8, 8),
                                 pl.ds(pl.multiple_of(c0v[j], 128), 128)],
                        win_v.at[k * _L + j],
                        sem,
                    )

            def _drain(j, _):
                pltpu.make_async_copy(
                    x_hbm.at[pl.ds(0, 8), pl.ds(0, 128)], win_v.at[0], sem
                ).wait()
                return 0

            lax.fori_loop(0, half, _drain, 0)

            for k in range(half // _L):
                rows = k * _L + lax.iota(jnp.int32, _L)
                subs = lax.iota(jnp.int32, _L) & 7
                tv = tgt_v[pl.ds(p * half + k * _L, _L)]
                lanes = tv & 127
                g_v[pl.ds(p * half + k * _L, _L)] = plsc.load_gather(
                    win_v, [rows, subs, lanes]
                )
        pltpu.sync_copy(g_v, g_hbm.at[pl.ds(row0, _BPW)])

    return _sc_gather


_BC = 640
_KC = _BC // 128
_NBLK = _SIZE // _BC


def _tc_body(x_ref, out_ref, acc_ref):
    j = pl.program_id(0)

    @pl.when(j == 0)
    def _init():
        acc_ref[...] = jnp.zeros_like(acc_ref)

    acc = acc_ref[...]
    for k in range(_KC):
        chunk = x_ref[:, k * 128:(k + 1) * 128]
        if k == 0:
            lane = lax.broadcasted_iota(jnp.int32, (_N, 128), 1)
            chunk = jnp.where((j == 0) & (lane == 0), 0.0, chunk)
        acc = acc + chunk
    acc_ref[...] = acc

    @pl.when(j == _NBLK - 1)
    def _finish():
        out_ref[...] = jnp.sum(acc_ref[...], axis=1, keepdims=True)


_tc_reduce = pl.pallas_call(
    _tc_body,
    grid=(_NBLK,),
    in_specs=[pl.BlockSpec((_N, _BC), lambda j: (0, j))],
    out_specs=pl.BlockSpec((_N, 1), lambda j: (0, 0)),
    out_shape=jax.ShapeDtypeStruct((_N, 1), jnp.float32),
    scratch_shapes=[pltpu.VMEM((_N, 128), jnp.float32)],
)


def _combine_body(tcs_ref, g_ref, t_ref, out_ref):
    rowsum = tcs_ref[...]
    g = g_ref[...]
    valid = t_ref[...] != _PAD
    li = _K0 - _EPS * (rowsum - g) - _CONF * g
    out_ref[0, 0] = jnp.sum(jnp.where(valid, li, 0.0))


_combine = pl.pallas_call(
    _combine_body,
    in_specs=[
        pl.BlockSpec((_N, 1), lambda: (0, 0)),
        pl.BlockSpec((_N, 1), lambda: (0, 0)),
        pl.BlockSpec((_N, 1), lambda: (0, 0)),
    ],
    out_specs=pl.BlockSpec((1, 1), lambda: (0, 0), memory_space=pltpu.SMEM),
    out_shape=jax.ShapeDtypeStruct((1, 1), jnp.float32),
)


def kernel(x, target):
    g = _make_sc_gather()(x, target)
    tcs = _tc_reduce(x)
    loss = _combine(tcs, g.reshape(_N, 1), target.reshape(_N, 1))
    return loss.reshape(())


# speedup vs baseline: 1.0142x; 1.0016x over previous
import functools
import math

import jax
import jax.numpy as jnp
import numpy as np
from jax import lax
from jax.experimental import pallas as pl
from jax.experimental.pallas import tpu as pltpu
from jax.experimental.pallas import tpu_sc as plsc

_SIZE = 32000
_PAD = 0
_SMOOTHING = 0.1
_CONF = 1.0 - _SMOOTHING
_N = 4096

_EPS = float(np.float32(_SMOOTHING / (_SIZE - 2)))
_K0 = (_SIZE - 2) * _EPS * math.log(_EPS) + _CONF * math.log(_CONF)

_NC, _NS, _L = 2, 16, 16
_NW = _NC * _NS
_BPW = _N // _NW


@functools.lru_cache(maxsize=None)
def _make_sc_gather():
    mesh = plsc.VectorSubcoreMesh(
        core_axis_name="c", subcore_axis_name="s", num_cores=_NC, num_subcores=_NS
    )

    @functools.partial(
        pl.kernel,
        out_type=jax.ShapeDtypeStruct((_N,), jnp.float32),
        mesh=mesh,
        compiler_params=pltpu.CompilerParams(
            use_tc_tiling_on_sc=True, needs_layout_passes=False
        ),
        scratch_types=[
            pltpu.VMEM((_BPW,), jnp.int32),
            pltpu.VMEM((_BPW // 2, 8, 128), jnp.float32),
            pltpu.VMEM((_BPW,), jnp.float32),
            pltpu.SemaphoreType.DMA,
        ],
    )
    def _sc_gather(x_hbm, tgt_hbm, g_hbm, tgt_v, win_v, g_v, sem):
        wid = lax.axis_index("s") * _NC + lax.axis_index("c")
        row0 = wid * _BPW
        pltpu.sync_copy(tgt_hbm.at[pl.ds(row0, _BPW)], tgt_v)

        half = _BPW // 2
        for p in range(2):
            for k in range(half // _L):
                tv = tgt_v[pl.ds(p * half + k * _L, _L)]
                c0v = (tv >> 7) << 7
                for j in range(_L):
                    r = p * half + k * _L + j
                    pltpu.async_copy(
                        x_hbm.at[pl.ds(row0 + (r // 8) * 8, 8),
                                 pl.ds(pl.multiple_of(c0v[j], 128), 128)],
                        win_v.at[k * _L + j],
                        sem,
                    )

            def _drain(j, _):
                pltpu.make_async_copy(
                    x_hbm.at[pl.ds(0, 8), pl.ds(0, 128)], win_v.at[0], sem
                ).wait()
                return 0

            lax.fori_loop(0, half, _drain, 0)

            for k in range(half // _L):
                rows = k * _L + lax.iota(jnp.int32, _L)
                subs = lax.iota(jnp.int32, _L) & 7
                tv = tgt_v[pl.ds(p * half + k * _L, _L)]
                lanes = tv & 127
                g_v[pl.ds(p * half + k * _L, _L)] = plsc.load_gather(
                    win_v, [rows, subs, lanes]
                )
        pltpu.sync_copy(g_v, g_hbm.at[pl.ds(row0, _BPW)])

    return _sc_gather


_BC = 640
_KC = _BC // 128
_NBLK = _SIZE // _BC


def _tc_body(x_ref, out_ref, acc_ref):
    j = pl.program_id(0)

    @pl.when(j == 0)
    def _init():
        acc_ref[...] = jnp.zeros_like(acc_ref)

    acc = acc_ref[...]
    for k in range(_KC):
        chunk = x_ref[:, k * 128:(k + 1) * 128]
        if k == 0:
            lane = lax.broadcasted_iota(jnp.int32, (_N, 128), 1)
            chunk = jnp.where((j == 0) & (lane == 0), 0.0, chunk)
        acc = acc + chunk
    acc_ref[...] = acc

    @pl.when(j == _NBLK - 1)
    def _finish():
        out_ref[...] = jnp.sum(acc_ref[...], axis=1, keepdims=True)


_tc_reduce = pl.pallas_call(
    _tc_body,
    grid=(_NBLK,),
    in_specs=[pl.BlockSpec((_N, _BC), lambda j: (0, j))],
    out_specs=pl.BlockSpec((_N, 1), lambda j: (0, 0)),
    out_shape=jax.ShapeDtypeStruct((_N, 1), jnp.float32),
    scratch_shapes=[pltpu.VMEM((_N, 128), jnp.float32)],
)


def _combine_body(tcs_ref, g_ref, t_ref, out_ref):
    rowsum = tcs_ref[...]
    g = g_ref[...]
    valid = t_ref[...] != _PAD
    li = _K0 - _EPS * (rowsum - g) - _CONF * g
    out_ref[0, 0] = jnp.sum(jnp.where(valid, li, 0.0))


_combine = pl.pallas_call(
    _combine_body,
    in_specs=[
        pl.BlockSpec((_N, 1), lambda: (0, 0)),
        pl.BlockSpec((_N, 1), lambda: (0, 0)),
        pl.BlockSpec((_N, 1), lambda: (0, 0)),
    ],
    out_specs=pl.BlockSpec((1, 1), lambda: (0, 0), memory_space=pltpu.SMEM),
    out_shape=jax.ShapeDtypeStruct((1, 1), jnp.float32),
)


def kernel(x, target):
    tcs = _tc_reduce(x)
    g = _make_sc_gather()(x, target)
    loss = _combine(tcs, g.reshape(_N, 1), target.reshape(_N, 1))
    return loss.reshape(())
